# baseline (speedup 1.0000x reference)
import functools

import jax
import jax.numpy as jnp
from jax import lax
from jax.experimental import pallas as pl
from jax.experimental.pallas import tpu as pltpu
from jax.experimental.pallas import tpu_sc as plsc

N_TX, N_BD, D_IN, HEADS, C_OUT = 10000, 1000, 128, 4, 32
D_HID = HEADS * C_OUT
SLOPE = 0.2
LANES = 16
NC, NS = 2, 16
NW = NC * NS
CH = 32
VREGS = D_HID // LANES

E_NN_PAD = 321536
E_SM_PAD = 10240
N_TX_PAD = 10240
N_BD_PAD = 1024


def _dn_rows(n_pad):
    return -(-(n_pad * HEADS // D_HID) // (NS * 8)) * NS * 8


def _edge_phase(n_dst_pad, e_pad):
    ept = e_pad // NW
    n_iters = ept // CH
    assert n_iters % 2 == 0
    rows_pt = n_dst_pad // NS
    dn_rows = _dn_rows(n_dst_pad)
    dn_pt = dn_rows // NS
    mesh = plsc.VectorSubcoreMesh(core_axis_name="c", subcore_axis_name="s")

    idx_t = pltpu.VMEM((CH,), jnp.int32)
    row_t = pltpu.VMEM((CH, D_HID), jnp.float32)

    @functools.partial(
        pl.kernel,
        mesh=mesh,
        out_type=(
            jax.ShapeDtypeStruct((NC, n_dst_pad, D_HID), jnp.float32),
            jax.ShapeDtypeStruct((NC, dn_rows, D_HID), jnp.float32),
        ),
        scratch_types=[
            [idx_t] * 2,
            [idx_t] * 2,
            [idx_t] * 2,
            [idx_t] * 2,
            [idx_t] * 2,
            [row_t] * 2,
            [row_t] * 2,
            [row_t] * 2,
            [row_t] * 2,
            pltpu.VMEM((D_HID,), jnp.float32),
            pltpu.VMEM_SHARED((n_dst_pad, D_HID), jnp.float32),
            pltpu.VMEM_SHARED((dn_rows, D_HID), jnp.float32),
            [pltpu.SemaphoreType.DMA] * 2,
            [pltpu.SemaphoreType.DMA] * 2,
            [pltpu.SemaphoreType.DMA] * 2,
        ],
    )
    def k(xl_hbm, xr_hbm, src_hbm, dstg_hbm, dsts_hbm, att_hbm,
          out_hbm, den_hbm,
          src_v, dstg_v, dsts_v, dstsc_v, dnidx_v, xl_v, xr_v, msg_v,
          den_v, att_v, acc_sh, accd_sh, isem, gsem, ssem):
        cid = lax.axis_index("c")
        sid = lax.axis_index("s")
        wid = sid * NC + cid

        zero = jnp.zeros((LANES,), jnp.float32)

        def zrow(i, carry):
            for j in range(VREGS):
                msg_v[0][i, pl.ds(j * LANES, LANES)] = zero
            return carry

        lax.fori_loop(0, CH, zrow, 0)

        row0 = sid * rows_pt
        off = 0
        while off < rows_pt:
            blk = min(CH, rows_pt - off)
            pltpu.sync_copy(msg_v[0].at[pl.ds(0, blk)],
                            acc_sh.at[pl.ds(row0 + off, blk)])
            off += blk
        off = 0
        while off < dn_pt:
            blk = min(CH, dn_pt - off)
            pltpu.sync_copy(msg_v[0].at[pl.ds(0, blk)],
                            accd_sh.at[pl.ds(sid * dn_pt + off, blk)])
            off += blk

        pltpu.sync_copy(att_hbm, att_v)
        plsc.subcore_barrier()

        att_r = [att_v[pl.ds(j * LANES, LANES)] for j in range(VREGS)]
        base = wid * ept

        def issue_idx(it, s):
            eoff = base + it * CH
            pltpu.async_copy(src_hbm.at[pl.ds(eoff, CH)], src_v[s], isem[s])
            pltpu.async_copy(dstg_hbm.at[pl.ds(eoff, CH)], dstg_v[s],
                             isem[s])
            pltpu.async_copy(dsts_hbm.at[pl.ds(eoff, CH)], dsts_v[s],
                             isem[s])

        def wait_idx(it, s):
            eoff = base + it * CH
            pltpu.make_async_copy(src_hbm.at[pl.ds(eoff, CH)], src_v[s],
                                  isem[s]).wait()
            pltpu.make_async_copy(dstg_hbm.at[pl.ds(eoff, CH)], dstg_v[s],
                                  isem[s]).wait()
            pltpu.make_async_copy(dsts_hbm.at[pl.ds(eoff, CH)], dsts_v[s],
                                  isem[s]).wait()

        HC = CH // 2

        def issue_gather(b):
            pltpu.async_copy(xl_hbm.at[src_v[b].at[pl.ds(0, HC)]],
                             xl_v[b].at[pl.ds(0, HC)], gsem[b])
            pltpu.async_copy(xr_hbm.at[dstg_v[b].at[pl.ds(0, HC)]],
                             xr_v[b].at[pl.ds(0, HC)], gsem[b])
            pltpu.async_copy(xl_hbm.at[src_v[b].at[pl.ds(HC, HC)]],
                             xl_v[b].at[pl.ds(HC, HC)], gsem[b])
            pltpu.async_copy(xr_hbm.at[dstg_v[b].at[pl.ds(HC, HC)]],
                             xr_v[b].at[pl.ds(HC, HC)], gsem[b])

        def wait_gather(b):
            for o in (0, HC):
                pltpu.make_async_copy(xl_hbm.at[src_v[b].at[pl.ds(o, HC)]],
                                      xl_v[b].at[pl.ds(o, HC)],
                                      gsem[b]).wait()
                pltpu.make_async_copy(xr_hbm.at[dstg_v[b].at[pl.ds(o, HC)]],
                                      xr_v[b].at[pl.ds(o, HC)],
                                      gsem[b]).wait()

        def issue_scatter(b):
            pltpu.async_copy(msg_v[b], acc_sh.at[dstsc_v[b]], ssem[b],
                             add=True)
            pltpu.async_copy(den_v[b], accd_sh.at[dnidx_v[b]], ssem[b],
                             add=True)

        def wait_scatter(b):
            pltpu.make_async_copy(msg_v[b], acc_sh.at[dstsc_v[b]],
                                  ssem[b]).wait()
            pltpu.make_async_copy(den_v[b], accd_sh.at[dnidx_v[b]],
                                  ssem[b]).wait()

        def compute(b):
            @plsc.parallel_loop(0, CH // LANES, 1)
            def group(g):
                dvec = dsts_v[b][pl.ds(g * LANES, LANES)]
                iot = lax.iota(jnp.int32, LANES)
                rot8 = (iot + 8) & 15
                rot4 = (iot + 4) & 15
                rot2 = (iot + 2) & 15
                rot1 = (iot + 1) & 15
                dstsc_v[b][pl.ds(g * LANES, LANES)] = dvec
                dnidx_v[b][pl.ds(g * LANES, LANES)] = dvec >> 5

                def rot(v, idx):
                    dn = lax.GatherDimensionNumbers(
                        offset_dims=(), collapsed_slice_dims=(0,),
                        start_index_map=(0,))
                    return lax.gather(
                        v, idx[:, None], dn, slice_sizes=(1,),
                        mode=lax.GatherScatterMode.PROMISE_IN_BOUNDS)

                for i2 in range(LANES):
                    i = g * LANES + i2
                    d = dvec[i2]
                    ebs = []
                    for h in range(HEADS):
                        xl0 = xl_v[b][i, pl.ds((2 * h) * LANES, LANES)]
                        xl1 = xl_v[b][i, pl.ds((2 * h + 1) * LANES, LANES)]
                        xr0 = xr_v[b][i, pl.ds((2 * h) * LANES, LANES)]
                        xr1 = xr_v[b][i, pl.ds((2 * h + 1) * LANES, LANES)]
                        t0 = xl0 + xr0
                        t1 = xl1 + xr1
                        z0 = jnp.maximum(t0, SLOPE * t0)
                        z1 = jnp.maximum(t1, SLOPE * t1)
                        q = z0 * att_r[2 * h] + z1 * att_r[2 * h + 1]
                        s8 = q + rot(q, rot8)
                        s4 = s8 + rot(s8, rot4)
                        s2 = s4 + rot(s4, rot2)
                        s1 = s2 + rot(s2, rot1)
                        eb = jnp.exp(s1)
                        ebs.append(eb)
                        msg_v[b][i, pl.ds((2 * h) * LANES, LANES)] = (
                            xl0 * eb)
                        msg_v[b][i, pl.ds((2 * h + 1) * LANES, LANES)] = (
                            xl1 * eb)
                    kv = (d >> 2) & 7
                    o4 = (d & 3) * HEADS
                    sh = jnp.where(
                        iot == o4, ebs[0],
                        jnp.where(iot == o4 + 1, ebs[1],
                                  jnp.where(iot == o4 + 2, ebs[2],
                                            jnp.where(iot == o4 + 3,
                                                      ebs[3], zero))))
                    for j in range(VREGS):
                        den_v[b][i, pl.ds(j * LANES, LANES)] = zero
                    den_v[b][i, pl.ds(kv * LANES, LANES)] = sh

        issue_idx(0, 0)
        issue_idx(1, 1)
        wait_idx(0, 0)
        issue_gather(0)

        def pair(itp, carry):
            for b in range(2):
                it = itp * 2 + b
                b2 = 1 - b

                @pl.when(it >= 2)
                def _():
                    wait_scatter(b)

                @pl.when(it + 1 < n_iters)
                def _():
                    wait_idx(it + 1, b2)
                    issue_gather(b2)

                wait_gather(b)

                compute(b)

                @pl.when(it + 2 < n_iters)
                def _():
                    issue_idx(it + 2, b)

                issue_scatter(b)
            return carry

        lax.fori_loop(0, n_iters // 2, pair, 0)
        wait_scatter(0)
        wait_scatter(1)
        plsc.subcore_barrier()

        off = 0
        while off < rows_pt:
            blk = min(CH, rows_pt - off)
            pltpu.sync_copy(acc_sh.at[pl.ds(row0 + off, blk)],
                            out_hbm.at[cid, pl.ds(row0 + off, blk)])
            off += blk
        off = 0
        while off < dn_pt:
            blk = min(CH, dn_pt - off)
            pltpu.sync_copy(accd_sh.at[pl.ds(sid * dn_pt + off, blk)],
                            den_hbm.at[cid, pl.ds(sid * dn_pt + off, blk)])
            off += blk

    return k


def _project(x, w_cat, n_out_mats, row_blk):
    n = x.shape[0]
    grid = pl.cdiv(n, row_blk)
    d_cat = w_cat.shape[0]

    def body(x_ref, w_ref, *outs):
        y = lax.dot_general(x_ref[...], w_ref[...], (((1,), (1,)), ((), ())),
                            preferred_element_type=jnp.float32)
        for m, o_ref in enumerate(outs):
            o_ref[...] = y[:, m * D_HID:(m + 1) * D_HID]

    return pl.pallas_call(
        body,
        grid=(grid,),
        in_specs=[
            pl.BlockSpec((row_blk, D_IN), lambda i: (i, 0)),
            pl.BlockSpec((d_cat, D_IN), lambda i: (0, 0)),
        ],
        out_specs=[pl.BlockSpec((row_blk, D_HID), lambda i: (i, 0))
                   for _ in range(n_out_mats)],
        out_shape=[jax.ShapeDtypeStruct((n, D_HID), jnp.float32)
                   for _ in range(n_out_mats)],
    )(x, w_cat)


def _combine(parts, bias, n_out, row_blk):
    grid = pl.cdiv(n_out, row_blk)
    n_parts = len(parts)

    def body(*refs):
        o_ref = refs[2 * n_parts + 1]
        b_ref = refs[2 * n_parts]
        acc = jnp.broadcast_to(b_ref[...], (row_blk, D_HID))
        col = lax.broadcasted_iota(jnp.int32, (HEADS, D_HID), 1)
        row = lax.broadcasted_iota(jnp.int32, (HEADS, D_HID), 0)
        sel = jnp.where(col // C_OUT == row, 1.0, 0.0)
        for p in range(n_parts):
            num_ref = refs[2 * p]
            den_ref = refs[2 * p + 1]
            numer = num_ref[0] + num_ref[1]
            den = den_ref[0] + den_ref[1]
            den_full = lax.dot_general(
                den, sel, (((1,), (0,)), ((), ())),
                preferred_element_type=jnp.float32)
            acc = acc + numer / (den_full + 1e-16)
        o_ref[...] = acc

    args = []
    in_specs = []
    for numer, den in parts:
        args += [numer, den]
        in_specs += [
            pl.BlockSpec((NC, row_blk, D_HID), lambda i: (0, i, 0)),
            pl.BlockSpec((NC, row_blk, HEADS), lambda i: (0, i, 0)),
        ]
    args.append(bias.reshape(1, D_HID))
    in_specs.append(pl.BlockSpec((1, D_HID), lambda i: (0, 0)))
    return pl.pallas_call(
        body,
        grid=(grid,),
        in_specs=in_specs,
        out_specs=pl.BlockSpec((row_blk, D_HID), lambda i: (i, 0)),
        out_shape=jax.ShapeDtypeStruct((n_out, D_HID), jnp.float32),
    )(*args)


def _pad_edges(ei, n_dst, e_pad):
    src = ei[0].astype(jnp.int32)
    dst = ei[1].astype(jnp.int32)
    pad = e_pad - src.shape[0]
    zpad = jnp.zeros((pad,), jnp.int32)
    src_p = jnp.concatenate([src, zpad])
    dst_g = jnp.concatenate([dst, zpad])
    dst_s = jnp.concatenate(
        [dst, n_dst + (jnp.arange(pad, dtype=jnp.int32) % LANES)])
    return src_p, dst_g, dst_s


def _den_view(den, n_pad):
    raw = n_pad * HEADS // D_HID
    return den[:, :raw].reshape(NC, n_pad, HEADS)


def kernel(x_tx, x_bd, ei_nn, ei_bl, ei_ct, Wl_nn, Wr_nn, att_nn, b_nn,
           Wl_bl, Wr_bl, att_bl, b_bl, Wl_ct, Wr_ct, att_ct, b_ct):
    w_tx = jnp.concatenate([Wl_nn, Wr_nn, Wr_ct, Wl_bl], axis=0)
    w_bd = jnp.concatenate([Wl_ct, Wr_bl], axis=0)
    xl_nn, xr_nn, xr_ct, xl_bl = _project(x_tx, w_tx, 4, 512)
    xl_ct, xr_bl = _project(x_bd, w_bd, 2, 504)

    src_nn, dstg_nn, dsts_nn = _pad_edges(ei_nn, N_TX, E_NN_PAD)
    src_bl, dstg_bl, dsts_bl = _pad_edges(ei_bl, N_BD, E_SM_PAD)
    src_ct, dstg_ct, dsts_ct = _pad_edges(ei_ct, N_TX, E_SM_PAD)

    ek_tx = _edge_phase(N_TX_PAD, E_NN_PAD)
    ek_tx_s = _edge_phase(N_TX_PAD, E_SM_PAD)
    ek_bd = _edge_phase(N_BD_PAD, E_SM_PAD)

    num_nn, den_nn = ek_tx(xl_nn, xr_nn, src_nn, dstg_nn, dsts_nn,
                           att_nn.reshape(D_HID))
    num_ct, den_ct = ek_tx_s(xl_ct, xr_ct, src_ct, dstg_ct, dsts_ct,
                             att_ct.reshape(D_HID))
    num_bl, den_bl = ek_bd(xl_bl, xr_bl, src_bl, dstg_bl, dsts_bl,
                           att_bl.reshape(D_HID))

    out_tx = _combine(
        [(num_nn, _den_view(den_nn, N_TX_PAD)),
         (num_ct, _den_view(den_ct, N_TX_PAD))],
        b_nn + b_ct, N_TX, 400)
    out_bd = _combine([(num_bl, _den_view(den_bl, N_BD_PAD))],
                      b_bl, N_BD, 200)
    return (out_tx, out_bd)


# speedup vs baseline: 1.8310x; 1.8310x over previous
import functools

import jax
import jax.numpy as jnp
from jax import lax
from jax.experimental import pallas as pl
from jax.experimental.pallas import tpu as pltpu
from jax.experimental.pallas import tpu_sc as plsc

N_TX, N_BD, D_IN, HEADS, C_OUT = 10000, 1000, 128, 4, 32
D_HID = HEADS * C_OUT
SLOPE = 0.2
LANES = 16
NC, NS = 2, 16
NW = NC * NS
CH = 32
VREGS = D_HID // LANES

E_NN_PAD = 321536
E_SM_PAD = 10240
N_TX_PAD = 10240
N_BD_PAD = 1024


def _dn_rows(n_pad):
    return -(-(n_pad * HEADS // D_HID) // (NS * 8)) * NS * 8


def _edge_phase(n_dst_pad, e_pad):
    ept = e_pad // NW
    n_iters = ept // CH
    assert n_iters % 2 == 0
    rows_pt = n_dst_pad // NS
    dn_rows = _dn_rows(n_dst_pad)
    dn_pt = dn_rows // NS
    mesh = plsc.VectorSubcoreMesh(core_axis_name="c", subcore_axis_name="s")

    idx_t = pltpu.VMEM((CH,), jnp.int32)
    row_t = pltpu.VMEM((CH, D_HID), jnp.float32)

    @functools.partial(
        pl.kernel,
        mesh=mesh,
        out_type=(
            jax.ShapeDtypeStruct((NC, n_dst_pad, D_HID), jnp.float32),
            jax.ShapeDtypeStruct((NC, dn_rows, D_HID), jnp.float32),
        ),
        scratch_types=[
            [idx_t] * 2,
            [idx_t] * 2,
            [pltpu.VMEM((CH + LANES,), jnp.int32)] * 2,
            [idx_t] * 2,
            [idx_t] * 2,
            [row_t] * 2,
            [row_t] * 2,
            [row_t] * 2,
            [row_t] * 2,
            pltpu.VMEM((D_HID,), jnp.float32),
            pltpu.VMEM_SHARED((n_dst_pad, D_HID), jnp.float32),
            pltpu.VMEM_SHARED((dn_rows, D_HID), jnp.float32),
            [pltpu.SemaphoreType.DMA] * 2,
            [pltpu.SemaphoreType.DMA] * 2,
            [pltpu.SemaphoreType.DMA] * 2,
        ],
    )
    def k(xl_hbm, xr_hbm, src_hbm, dstg_hbm, dsts_hbm, att_hbm,
          out_hbm, den_hbm,
          src_v, dstg_v, dsts_v, dstsc_v, dnidx_v, xl_v, xr_v, msg_v,
          den_v, att_v, acc_sh, accd_sh, isem, gsem, ssem):
        cid = lax.axis_index("c")
        sid = lax.axis_index("s")
        wid = sid * NC + cid

        zero = jnp.zeros((LANES,), jnp.float32)

        def zrow(i, carry):
            for j in range(VREGS):
                msg_v[0][i, pl.ds(j * LANES, LANES)] = zero
            return carry

        lax.fori_loop(0, CH, zrow, 0)

        row0 = sid * rows_pt
        off = 0
        while off < rows_pt:
            blk = min(CH, rows_pt - off)
            pltpu.sync_copy(msg_v[0].at[pl.ds(0, blk)],
                            acc_sh.at[pl.ds(row0 + off, blk)])
            off += blk
        off = 0
        while off < dn_pt:
            blk = min(CH, dn_pt - off)
            pltpu.sync_copy(msg_v[0].at[pl.ds(0, blk)],
                            accd_sh.at[pl.ds(sid * dn_pt + off, blk)])
            off += blk

        pltpu.sync_copy(att_hbm, att_v)
        plsc.subcore_barrier()

        att_r = [att_v[pl.ds(j * LANES, LANES)] for j in range(VREGS)]
        base = wid * ept

        def issue_idx(it, s):
            eoff = base + it * CH
            pltpu.async_copy(src_hbm.at[pl.ds(eoff, CH)], src_v[s], isem[s])
            pltpu.async_copy(dstg_hbm.at[pl.ds(eoff, CH)], dstg_v[s],
                             isem[s])
            pltpu.async_copy(dsts_hbm.at[pl.ds(eoff, CH)],
                             dsts_v[s].at[pl.ds(0, CH)], isem[s])

        def wait_idx(it, s):
            eoff = base + it * CH
            pltpu.make_async_copy(src_hbm.at[pl.ds(eoff, CH)], src_v[s],
                                  isem[s]).wait()
            pltpu.make_async_copy(dstg_hbm.at[pl.ds(eoff, CH)], dstg_v[s],
                                  isem[s]).wait()
            pltpu.make_async_copy(dsts_hbm.at[pl.ds(eoff, CH)],
                                  dsts_v[s].at[pl.ds(0, CH)],
                                  isem[s]).wait()

        HC = CH // 2

        def issue_gather(b):
            pltpu.async_copy(xl_hbm.at[src_v[b].at[pl.ds(0, HC)]],
                             xl_v[b].at[pl.ds(0, HC)], gsem[b])
            pltpu.async_copy(xr_hbm.at[dstg_v[b].at[pl.ds(0, HC)]],
                             xr_v[b].at[pl.ds(0, HC)], gsem[b])
            pltpu.async_copy(xl_hbm.at[src_v[b].at[pl.ds(HC, HC)]],
                             xl_v[b].at[pl.ds(HC, HC)], gsem[b])
            pltpu.async_copy(xr_hbm.at[dstg_v[b].at[pl.ds(HC, HC)]],
                             xr_v[b].at[pl.ds(HC, HC)], gsem[b])

        def wait_gather(b):
            for o in (0, HC):
                pltpu.make_async_copy(xl_hbm.at[src_v[b].at[pl.ds(o, HC)]],
                                      xl_v[b].at[pl.ds(o, HC)],
                                      gsem[b]).wait()
                pltpu.make_async_copy(xr_hbm.at[dstg_v[b].at[pl.ds(o, HC)]],
                                      xr_v[b].at[pl.ds(o, HC)],
                                      gsem[b]).wait()

        def issue_scatter(b):
            pltpu.async_copy(msg_v[b], acc_sh.at[dstsc_v[b]], ssem[b],
                             add=True)
            pltpu.async_copy(den_v[b], accd_sh.at[dnidx_v[b]], ssem[b],
                             add=True)

        def wait_scatter(b):
            pltpu.make_async_copy(msg_v[b], acc_sh.at[dstsc_v[b]],
                                  ssem[b]).wait()
            pltpu.make_async_copy(den_v[b], accd_sh.at[dnidx_v[b]],
                                  ssem[b]).wait()

        def compute(b):
            def gidx(g, c0):
                dvec = dsts_v[b][pl.ds(g * LANES, LANES)]
                dstsc_v[b][pl.ds(g * LANES, LANES)] = dvec
                dnidx_v[b][pl.ds(g * LANES, LANES)] = dvec >> 5
                return c0

            lax.fori_loop(0, CH // LANES, gidx, 0)

            @plsc.parallel_loop(0, CH, 1, unroll=2)
            def edge(i):
                iot = lax.iota(jnp.int32, LANES)
                rot8 = (iot + 8) & 15
                rot4 = (iot + 4) & 15
                rot2 = (iot + 2) & 15
                rot1 = (iot + 1) & 15

                def rot(v, idx):
                    dn = lax.GatherDimensionNumbers(
                        offset_dims=(), collapsed_slice_dims=(0,),
                        start_index_map=(0,))
                    return lax.gather(
                        v, idx[:, None], dn, slice_sizes=(1,),
                        mode=lax.GatherScatterMode.PROMISE_IN_BOUNDS)

                d = dsts_v[b][pl.ds(i, LANES)][0]
                ebs = []
                for h in range(HEADS):
                    xl0 = xl_v[b][i, pl.ds((2 * h) * LANES, LANES)]
                    xl1 = xl_v[b][i, pl.ds((2 * h + 1) * LANES, LANES)]
                    xr0 = xr_v[b][i, pl.ds((2 * h) * LANES, LANES)]
                    xr1 = xr_v[b][i, pl.ds((2 * h + 1) * LANES, LANES)]
                    t0 = xl0 + xr0
                    t1 = xl1 + xr1
                    z0 = jnp.maximum(t0, SLOPE * t0)
                    z1 = jnp.maximum(t1, SLOPE * t1)
                    q = z0 * att_r[2 * h] + z1 * att_r[2 * h + 1]
                    s8 = q + rot(q, rot8)
                    s4 = s8 + rot(s8, rot4)
                    s2 = s4 + rot(s4, rot2)
                    s1 = s2 + rot(s2, rot1)
                    eb = jnp.exp(s1)
                    ebs.append(eb)
                    msg_v[b][i, pl.ds((2 * h) * LANES, LANES)] = xl0 * eb
                    msg_v[b][i, pl.ds((2 * h + 1) * LANES, LANES)] = (
                        xl1 * eb)
                kv = (d >> 2) & 7
                o4 = (d & 3) * HEADS
                sh = jnp.where(
                    iot == o4, ebs[0],
                    jnp.where(iot == o4 + 1, ebs[1],
                              jnp.where(iot == o4 + 2, ebs[2],
                                        jnp.where(iot == o4 + 3,
                                                  ebs[3], zero))))
                for j in range(VREGS):
                    den_v[b][i, pl.ds(j * LANES, LANES)] = zero
                den_v[b][i, pl.ds(kv * LANES, LANES)] = sh

        issue_idx(0, 0)
        issue_idx(1, 1)
        wait_idx(0, 0)
        issue_gather(0)

        def pair(itp, carry):
            for b in range(2):
                it = itp * 2 + b
                b2 = 1 - b

                @pl.when(it >= 2)
                def _():
                    wait_scatter(b)

                @pl.when(it + 1 < n_iters)
                def _():
                    wait_idx(it + 1, b2)
                    issue_gather(b2)

                wait_gather(b)

                compute(b)

                @pl.when(it + 2 < n_iters)
                def _():
                    issue_idx(it + 2, b)

                issue_scatter(b)
            return carry

        lax.fori_loop(0, n_iters // 2, pair, 0)
        wait_scatter(0)
        wait_scatter(1)
        plsc.subcore_barrier()

        off = 0
        while off < rows_pt:
            blk = min(CH, rows_pt - off)
            pltpu.sync_copy(acc_sh.at[pl.ds(row0 + off, blk)],
                            out_hbm.at[cid, pl.ds(row0 + off, blk)])
            off += blk
        off = 0
        while off < dn_pt:
            blk = min(CH, dn_pt - off)
            pltpu.sync_copy(accd_sh.at[pl.ds(sid * dn_pt + off, blk)],
                            den_hbm.at[cid, pl.ds(sid * dn_pt + off, blk)])
            off += blk

    return k


def _project(x, w_cat, n_out_mats, row_blk):
    n = x.shape[0]
    grid = pl.cdiv(n, row_blk)
    d_cat = w_cat.shape[0]

    def body(x_ref, w_ref, *outs):
        y = lax.dot_general(x_ref[...], w_ref[...], (((1,), (1,)), ((), ())),
                            preferred_element_type=jnp.float32)
        for m, o_ref in enumerate(outs):
            o_ref[...] = y[:, m * D_HID:(m + 1) * D_HID]

    return pl.pallas_call(
        body,
        grid=(grid,),
        in_specs=[
            pl.BlockSpec((row_blk, D_IN), lambda i: (i, 0)),
            pl.BlockSpec((d_cat, D_IN), lambda i: (0, 0)),
        ],
        out_specs=[pl.BlockSpec((row_blk, D_HID), lambda i: (i, 0))
                   for _ in range(n_out_mats)],
        out_shape=[jax.ShapeDtypeStruct((n, D_HID), jnp.float32)
                   for _ in range(n_out_mats)],
    )(x, w_cat)


def _combine(parts, bias, n_out, row_blk):
    grid = pl.cdiv(n_out, row_blk)
    n_parts = len(parts)

    def body(*refs):
        o_ref = refs[2 * n_parts + 1]
        b_ref = refs[2 * n_parts]
        acc = jnp.broadcast_to(b_ref[...], (row_blk, D_HID))
        col = lax.broadcasted_iota(jnp.int32, (HEADS, D_HID), 1)
        row = lax.broadcasted_iota(jnp.int32, (HEADS, D_HID), 0)
        sel = jnp.where(col // C_OUT == row, 1.0, 0.0)
        for p in range(n_parts):
            num_ref = refs[2 * p]
            den_ref = refs[2 * p + 1]
            numer = num_ref[0] + num_ref[1]
            den = den_ref[0] + den_ref[1]
            den_full = lax.dot_general(
                den, sel, (((1,), (0,)), ((), ())),
                preferred_element_type=jnp.float32)
            acc = acc + numer / (den_full + 1e-16)
        o_ref[...] = acc

    args = []
    in_specs = []
    for numer, den in parts:
        args += [numer, den]
        in_specs += [
            pl.BlockSpec((NC, row_blk, D_HID), lambda i: (0, i, 0)),
            pl.BlockSpec((NC, row_blk, HEADS), lambda i: (0, i, 0)),
        ]
    args.append(bias.reshape(1, D_HID))
    in_specs.append(pl.BlockSpec((1, D_HID), lambda i: (0, 0)))
    return pl.pallas_call(
        body,
        grid=(grid,),
        in_specs=in_specs,
        out_specs=pl.BlockSpec((row_blk, D_HID), lambda i: (i, 0)),
        out_shape=jax.ShapeDtypeStruct((n_out, D_HID), jnp.float32),
    )(*args)


def _pad_edges(ei, n_dst, e_pad):
    src = ei[0].astype(jnp.int32)
    dst = ei[1].astype(jnp.int32)
    pad = e_pad - src.shape[0]
    zpad = jnp.zeros((pad,), jnp.int32)
    src_p = jnp.concatenate([src, zpad])
    dst_g = jnp.concatenate([dst, zpad])
    dst_s = jnp.concatenate(
        [dst, n_dst + (jnp.arange(pad, dtype=jnp.int32) % LANES)])
    return src_p, dst_g, dst_s


def _den_view(den, n_pad):
    raw = n_pad * HEADS // D_HID
    return den[:, :raw].reshape(NC, n_pad, HEADS)


def kernel(x_tx, x_bd, ei_nn, ei_bl, ei_ct, Wl_nn, Wr_nn, att_nn, b_nn,
           Wl_bl, Wr_bl, att_bl, b_bl, Wl_ct, Wr_ct, att_ct, b_ct):
    w_tx = jnp.concatenate([Wl_nn, Wr_nn, Wr_ct, Wl_bl], axis=0)
    w_bd = jnp.concatenate([Wl_ct, Wr_bl], axis=0)
    xl_nn, xr_nn, xr_ct, xl_bl = _project(x_tx, w_tx, 4, 512)
    xl_ct, xr_bl = _project(x_bd, w_bd, 2, 504)

    src_nn, dstg_nn, dsts_nn = _pad_edges(ei_nn, N_TX, E_NN_PAD)
    src_bl, dstg_bl, dsts_bl = _pad_edges(ei_bl, N_BD, E_SM_PAD)
    src_ct, dstg_ct, dsts_ct = _pad_edges(ei_ct, N_TX, E_SM_PAD)

    ek_tx = _edge_phase(N_TX_PAD, E_NN_PAD)
    ek_tx_s = _edge_phase(N_TX_PAD, E_SM_PAD)
    ek_bd = _edge_phase(N_BD_PAD, E_SM_PAD)

    num_nn, den_nn = ek_tx(xl_nn, xr_nn, src_nn, dstg_nn, dsts_nn,
                           att_nn.reshape(D_HID))
    num_ct, den_ct = ek_tx_s(xl_ct, xr_ct, src_ct, dstg_ct, dsts_ct,
                             att_ct.reshape(D_HID))
    num_bl, den_bl = ek_bd(xl_bl, xr_bl, src_bl, dstg_bl, dsts_bl,
                           att_bl.reshape(D_HID))

    out_tx = _combine(
        [(num_nn, _den_view(den_nn, N_TX_PAD)),
         (num_ct, _den_view(den_ct, N_TX_PAD))],
        b_nn + b_ct, N_TX, 400)
    out_bd = _combine([(num_bl, _den_view(den_bl, N_BD_PAD))],
                      b_bl, N_BD, 200)
    return (out_tx, out_bd)
